# Initial kernel scaffold; baseline (speedup 1.0000x reference)
import jax
import jax.numpy as jnp
from jax.experimental import pallas as pl


def kernel(node_feat_input, adjacency_input, indices, W_ih, W_hh, b_ih, b_hh, W_out, b_out):
    raise NotImplementedError("write your pallas kernel here")


# speedup vs baseline: 11.5609x; 11.5609x over previous
import functools

import jax
import jax.numpy as jnp
from jax import lax
from jax.experimental import pallas as pl
from jax.experimental.pallas import tpu as pltpu
from jax.experimental.pallas import tpu_sc as plsc

N = 4096
E = 65536
F = 256
OUT = 128
KARY = 5

NC = 2
NS = 16
L = 16
HALF = N // NC
NPT = HALF // NS
CHUNK = E // NS


def _sc_body(src_hbm, dst_hbm, x_hbm, feats_hbm, len_hbm,
             src_v, dst_v, cnt_v, loc_v, cnt_t, loc_t, selT, len_t, rows_v,
             counts_sh, locs_sh, sem):
    cid = lax.axis_index("c")
    sid = lax.axis_index("s")
    lo = cid * HALF

    pltpu.sync_copy(src_hbm.at[pl.ds(sid * CHUNK, CHUNK)], src_v)
    pltpu.sync_copy(dst_hbm.at[pl.ds(sid * CHUNK, CHUNK)], dst_v)

    @plsc.parallel_loop(0, HALF, step=L)
    def _(i):
        cnt_v[pl.ds(i, L)] = jnp.zeros((L,), jnp.int32)

    def edge_body(i, _):
        s16 = src_v[pl.ds(i * L, L)]
        d16 = dst_v[pl.ds(i * L, L)]
        sl_raw = s16 - lo
        valid = (sl_raw >= 0) & (sl_raw < HALF)
        occ, lastm = plsc.scan_count(sl_raw, mask=valid)
        sl = sl_raw & (HALF - 1)
        cnt16 = plsc.load_gather(cnt_v, [sl], mask=valid)
        p = cnt16 + occ - 1
        plsc.store_scatter(loc_v, [sl * KARY + p], d16,
                           mask=valid & (p < KARY))
        plsc.store_scatter(cnt_v, [sl], cnt16 + occ, mask=valid & lastm)
        return 0

    lax.fori_loop(0, CHUNK // L, edge_body, 0)

    pltpu.sync_copy(cnt_v, counts_sh.at[sid])
    pltpu.sync_copy(loc_v, locs_sh.at[sid])
    plsc.subcore_barrier()

    nb = sid * NPT
    pltpu.sync_copy(counts_sh.at[:, pl.ds(nb, NPT)], cnt_t)
    pltpu.sync_copy(locs_sh.at[:, pl.ds(nb * KARY, NPT * KARY)], loc_t)

    @plsc.parallel_loop(0, NPT * KARY, step=L)
    def _(i):
        selT[pl.ds(i, L)] = jnp.zeros((L,), jnp.int32)

    @plsc.parallel_loop(0, NPT, step=L)
    def _(j):
        acc = jnp.zeros((L,), jnp.int32)
        for w in range(NS):
            acc = acc + jnp.minimum(cnt_t[w, pl.ds(j, L)], KARY)
        len_t[pl.ds(j, L)] = jnp.minimum(acc, KARY)

    lanes = lax.iota(jnp.int32, L)

    def merge_body(j, _):
        jv = jnp.full((L,), j, jnp.int32)
        col = plsc.load_gather(cnt_t, [lanes, jv])
        cc = jnp.minimum(col, KARY)
        off = plsc.cumsum(cc) - cc
        for k in range(KARY):
            vals = plsc.load_gather(loc_t, [lanes, jv * KARY + k])
            pos = off + k
            m = (k < cc) & (pos < KARY)
            plsc.store_scatter(selT, [pos * NPT + j], vals, mask=m)
        return 0

    lax.fori_loop(0, NPT, merge_body, 0)

    pltpu.sync_copy(len_t, len_hbm.at[pl.ds(lo + nb, NPT)])

    for t in range(KARY):
        pltpu.async_copy(x_hbm.at[selT.at[pl.ds(t * NPT, NPT)]], rows_v,
                         sem).wait()
        pltpu.sync_copy(rows_v, feats_hbm.at[t, pl.ds(lo + nb, NPT)])


def _sc_build_gather(src, dst, x):
    mesh = plsc.VectorSubcoreMesh(core_axis_name="c", subcore_axis_name="s")
    kern = pl.kernel(
        _sc_body,
        out_type=[
            jax.ShapeDtypeStruct((KARY, N, F), jnp.float32),
            jax.ShapeDtypeStruct((N,), jnp.int32),
        ],
        mesh=mesh,
        scratch_types=[
            pltpu.VMEM((CHUNK,), jnp.int32),
            pltpu.VMEM((CHUNK,), jnp.int32),
            pltpu.VMEM((HALF,), jnp.int32),
            pltpu.VMEM((HALF * KARY,), jnp.int32),
            pltpu.VMEM((NS, NPT), jnp.int32),
            pltpu.VMEM((NS, NPT * KARY), jnp.int32),
            pltpu.VMEM((NPT * KARY,), jnp.int32),
            pltpu.VMEM((NPT,), jnp.int32),
            pltpu.VMEM((NPT, F), jnp.float32),
            pltpu.VMEM_SHARED((NS, HALF), jnp.int32),
            pltpu.VMEM_SHARED((NS, HALF * KARY), jnp.int32),
            pltpu.SemaphoreType.DMA,
        ],
        compiler_params=pltpu.CompilerParams(needs_layout_passes=False),
    )
    return kern(src, dst, x)


def _tc_body(x_ref, feats_ref, len_ref, wih_ref, whh_ref, bih_ref, bhh_ref,
             wox_ref, woc_ref, bout_ref, out_ref):
    xb = x_ref[...]
    b = bih_ref[...] + bhh_ref[...]
    lens = len_ref[...]
    blk = xb.shape[0]
    c = jnp.zeros((blk, F), jnp.float32)
    h = jnp.zeros((blk, F), jnp.float32)
    for t in range(KARY):
        ft = feats_ref[t]
        gates = jnp.dot(ft, wih_ref[...], preferred_element_type=jnp.float32) + b
        if t > 0:
            gates = gates + jnp.dot(h, whh_ref[...],
                                    preferred_element_type=jnp.float32)
        i_g = gates[:, :F]
        f_g = gates[:, F:2 * F]
        g_g = gates[:, 2 * F:3 * F]
        o_g = gates[:, 3 * F:]
        c_new = jax.nn.sigmoid(f_g) * c + jax.nn.sigmoid(i_g) * jnp.tanh(g_g)
        h_new = jax.nn.sigmoid(o_g) * jnp.tanh(c_new)
        m = t < lens
        c = jnp.where(m, c_new, c)
        h = jnp.where(m, h_new, h)
    y = (jnp.dot(xb, wox_ref[...], preferred_element_type=jnp.float32)
         + jnp.dot(c, woc_ref[...], preferred_element_type=jnp.float32)
         + bout_ref[...])
    out_ref[...] = jax.nn.sigmoid(y)


def _tc_lstm(x, feats, lens, wihT, whhT, bih, bhh, wox, woc, bout):
    B = 512
    grid = (N // B,)
    return pl.pallas_call(
        _tc_body,
        grid=grid,
        in_specs=[
            pl.BlockSpec((B, F), lambda i: (i, 0)),
            pl.BlockSpec((KARY, B, F), lambda i: (0, i, 0)),
            pl.BlockSpec((B, 1), lambda i: (i, 0)),
            pl.BlockSpec((F, 4 * F), lambda i: (0, 0)),
            pl.BlockSpec((F, 4 * F), lambda i: (0, 0)),
            pl.BlockSpec((1, 4 * F), lambda i: (0, 0)),
            pl.BlockSpec((1, 4 * F), lambda i: (0, 0)),
            pl.BlockSpec((F, OUT), lambda i: (0, 0)),
            pl.BlockSpec((F, OUT), lambda i: (0, 0)),
            pl.BlockSpec((1, OUT), lambda i: (0, 0)),
        ],
        out_specs=pl.BlockSpec((B, OUT), lambda i: (i, 0)),
        out_shape=jax.ShapeDtypeStruct((N, OUT), jnp.float32),
        compiler_params=pltpu.CompilerParams(
            dimension_semantics=("arbitrary",)),
    )(x, feats, lens, wihT, whhT, bih, bhh, wox, woc, bout)


def kernel(node_feat_input, adjacency_input, indices, W_ih, W_hh, b_ih, b_hh,
           W_out, b_out):
    src = adjacency_input[:, 0]
    dst = adjacency_input[:, 1]
    feats, lengths = _sc_build_gather(src, dst, node_feat_input)
    return _tc_lstm(
        node_feat_input, feats, lengths.reshape(N, 1),
        W_ih.T, W_hh.T, b_ih.reshape(1, -1), b_hh.reshape(1, -1),
        W_out[:F], W_out[F:], b_out.reshape(1, -1))
